# baseline (speedup 1.0000x reference)
import functools

import jax
import jax.numpy as jnp
from jax import lax
from jax.experimental import pallas as pl
from jax.experimental.pallas import tpu as pltpu
from jax.experimental.pallas import tpu_sc as plsc

_NC = 2
_NS = 16
_NW = _NC * _NS

_CHUNK = 32
_NB = 4


@functools.lru_cache(maxsize=None)
def _build_embed(R: int, S: int, V: int, D: int):
    B = R * S
    assert B % _NW == 0
    b_per_w = B // _NW
    assert b_per_w % _CHUNK == 0
    assert S % b_per_w == 0
    nch = b_per_w // _CHUNK
    mesh = plsc.VectorSubcoreMesh(core_axis_name="c", subcore_axis_name="s")

    @functools.partial(
        pl.kernel,
        mesh=mesh,
        out_type=jax.ShapeDtypeStruct((R, S, D), jnp.float32),
        scratch_types=[
            pltpu.VMEM((b_per_w,), jnp.int32),
            pltpu.VMEM((_NB, _CHUNK, D), jnp.float32),
            pltpu.SemaphoreType.DMA,
            pltpu.SemaphoreType.DMA,
            pltpu.SemaphoreType.DMA,
            pltpu.SemaphoreType.DMA,
            pltpu.SemaphoreType.DMA,
            pltpu.SemaphoreType.DMA,
            pltpu.SemaphoreType.DMA,
            pltpu.SemaphoreType.DMA,
        ],
    )
    def embed_k(tok_hbm, table_hbm, out_hbm, idx_v, rows_v, *sems):
        gsem = sems[:_NB]
        osem = sems[_NB:]
        wid = lax.axis_index("s") * _NC + lax.axis_index("c")
        base = wid * b_per_w
        r = base // S
        s0 = base % S
        pltpu.sync_copy(tok_hbm.at[r, pl.ds(s0, b_per_w)], idx_v)

        def gather(j):
            return pltpu.async_copy(
                table_hbm.at[idx_v.at[pl.ds(j * _CHUNK, _CHUNK)]],
                rows_v.at[j % _NB],
                gsem[j % _NB],
            )

        gh = [None] * nch
        oh = [None] * nch
        for j in range(min(_NB - 1, nch)):
            gh[j] = gather(j)
        for j in range(nch):
            b = j % _NB
            gh[j].wait()
            oh[j] = pltpu.async_copy(
                rows_v.at[b],
                out_hbm.at[r, pl.ds(s0 + j * _CHUNK, _CHUNK)],
                osem[b],
            )
            jn = j + _NB - 1
            if jn < nch:
                if j >= 1:
                    oh[j - 1].wait()
                gh[jn] = gather(jn)
        for j in range(max(0, nch - _NB), nch):
            oh[j].wait()

    return embed_k


def kernel(tokens, W_E):
    V, D = W_E.shape
    R, S = tokens.shape
    return _build_embed(R, S, V, D)(tokens.astype(jnp.int32), W_E)


# speedup vs baseline: 1.0041x; 1.0041x over previous
import functools

import jax
import jax.numpy as jnp
from jax import lax
from jax.experimental import pallas as pl
from jax.experimental.pallas import tpu as pltpu
from jax.experimental.pallas import tpu_sc as plsc

_NC = 2
_NS = 16
_NW = _NC * _NS

_CHUNK = 32
_NB = 5


@functools.lru_cache(maxsize=None)
def _build_embed(R: int, S: int, V: int, D: int):
    B = R * S
    assert B % _NW == 0
    b_per_w = B // _NW
    assert b_per_w % _CHUNK == 0
    assert S % b_per_w == 0
    nch = b_per_w // _CHUNK
    mesh = plsc.VectorSubcoreMesh(core_axis_name="c", subcore_axis_name="s")

    @functools.partial(
        pl.kernel,
        mesh=mesh,
        out_type=jax.ShapeDtypeStruct((R, S, D), jnp.float32),
        scratch_types=[
            pltpu.VMEM((b_per_w,), jnp.int32),
            pltpu.VMEM((_NB, _CHUNK, D), jnp.float32),
        ]
        + [pltpu.SemaphoreType.DMA] * (2 * _NB),
    )
    def embed_k(tok_hbm, table_hbm, out_hbm, idx_v, rows_v, *sems):
        gsem = sems[:_NB]
        osem = sems[_NB:]
        wid = lax.axis_index("s") * _NC + lax.axis_index("c")
        base = wid * b_per_w
        r = base // S
        s0 = base % S
        pltpu.sync_copy(tok_hbm.at[r, pl.ds(s0, b_per_w)], idx_v)

        def gather(j):
            return pltpu.async_copy(
                table_hbm.at[idx_v.at[pl.ds(j * _CHUNK, _CHUNK)]],
                rows_v.at[j % _NB],
                gsem[j % _NB],
            )

        gh = [None] * nch
        oh = [None] * nch
        for j in range(min(_NB - 1, nch)):
            gh[j] = gather(j)
        for j in range(nch):
            b = j % _NB
            gh[j].wait()
            oh[j] = pltpu.async_copy(
                rows_v.at[b],
                out_hbm.at[r, pl.ds(s0 + j * _CHUNK, _CHUNK)],
                osem[b],
            )
            jn = j + _NB - 1
            if jn < nch:
                if j >= 1:
                    oh[j - 1].wait()
                gh[jn] = gather(jn)
        for j in range(max(0, nch - _NB), nch):
            oh[j].wait()

    return embed_k


def kernel(tokens, W_E):
    V, D = W_E.shape
    R, S = tokens.shape
    return _build_embed(R, S, V, D)(tokens.astype(jnp.int32), W_E)
